# Initial kernel scaffold; baseline (speedup 1.0000x reference)
import jax
import jax.numpy as jnp
from jax.experimental import pallas as pl


def kernel(input, tok_table, pos_table):
    raise NotImplementedError("write your pallas kernel here")


# speedup vs baseline: 8.2108x; 8.2108x over previous
import functools

import jax
import jax.numpy as jnp
from jax import lax
from jax.experimental import pallas as pl
from jax.experimental.pallas import tpu as pltpu
from jax.experimental.pallas import tpu_sc as plsc

NC = 2
NS = 16
NW = NC * NS

BATCH = 1024
SEQ = 200
EMBED = 128
LANES = 16
CH = 40
NCH = SEQ // CH
SPW = BATCH // NW


def _emb_body(idx_hbm, tok_hbm, pos_hbm, out_hbm, pos_v, idx_v, row_v, sem):
    wid = lax.axis_index("s") * NC + lax.axis_index("c")

    pltpu.sync_copy(pos_hbm.at[pl.ds(0, SEQ)], pos_v)

    pltpu.sync_copy(idx_hbm.at[pl.ds(wid * SPW * NCH, SPW * NCH)], idx_v)

    def seq_body(s, carry):
        seq_id = wid * SPW + s
        base = seq_id * SEQ

        descs = [
            pltpu.async_copy(
                tok_hbm.at[idx_v.at[s * NCH + j]],
                row_v.at[pl.ds(j * CH, CH)],
                sem,
            )
            for j in range(NCH)
        ]
        for d in descs:
            d.wait()

        def add_body(i, c):
            for j in range(EMBED // LANES):
                sl = pl.ds(j * LANES, LANES)
                row_v[i, sl] = row_v[i, sl] + pos_v[i, sl]
            return c

        lax.fori_loop(0, SEQ, add_body, 0)

        pltpu.sync_copy(row_v, out_hbm.at[pl.ds(base, SEQ)])
        return carry

    lax.fori_loop(0, SPW, seq_body, 0)


def kernel(input, tok_table, pos_table):
    idx = input.astype(jnp.int32).reshape(BATCH * SEQ // CH, CH)

    mesh = plsc.VectorSubcoreMesh(
        core_axis_name="c", subcore_axis_name="s", num_cores=NC, num_subcores=NS
    )
    emb = functools.partial(
        pl.kernel,
        out_type=jax.ShapeDtypeStruct((BATCH * SEQ, EMBED), jnp.float32),
        mesh=mesh,
        scratch_types=[
            pltpu.VMEM((SEQ, EMBED), jnp.float32),
            pltpu.VMEM((SPW * NCH, CH), jnp.int32),
            pltpu.VMEM((SEQ, EMBED), jnp.float32),
            pltpu.SemaphoreType.DMA,
        ],
    )(_emb_body)
    out = emb(idx, tok_table, pos_table)
    return out.reshape(BATCH, SEQ, EMBED)


# speedup vs baseline: 11.8335x; 1.4412x over previous
import functools

import jax
import jax.numpy as jnp
from jax import lax
from jax.experimental import pallas as pl
from jax.experimental.pallas import tpu as pltpu
from jax.experimental.pallas import tpu_sc as plsc

NC = 2
NS = 16
NW = NC * NS

BATCH = 1024
SEQ = 200
EMBED = 128
LANES = 16
CH = 40
NCH = SEQ // CH
SPW = BATCH // NW


def _emb_body(idx_hbm, tok_hbm, pos_hbm, out_hbm,
              pos_v, idx_v, row_a, row_b,
              gsem_a, gsem_b, wsem_a, wsem_b):
    wid = lax.axis_index("s") * NC + lax.axis_index("c")

    pltpu.sync_copy(pos_hbm.at[pl.ds(0, SEQ)], pos_v)
    pltpu.sync_copy(idx_hbm.at[pl.ds(wid * SPW * NCH, SPW * NCH)], idx_v)

    def fire_gathers(s, buf, sem):
        for j in range(NCH):
            pltpu.async_copy(
                tok_hbm.at[idx_v.at[s * NCH + j]],
                buf.at[pl.ds(j * CH, CH)],
                sem,
            )

    def wait_gathers(buf, sem):
        pltpu.make_async_copy(tok_hbm.at[pl.ds(0, SEQ)], buf, sem).wait()

    def fire_writeout(s, buf, sem):
        base = (wid * SPW + s) * SEQ
        pltpu.async_copy(buf, out_hbm.at[pl.ds(base, SEQ)], sem)

    def wait_writeout(buf, sem):
        pltpu.make_async_copy(buf, out_hbm.at[pl.ds(0, SEQ)], sem).wait()

    def add_pos(buf):
        def body(i, c):
            for j in range(EMBED // LANES):
                sl = pl.ds(j * LANES, LANES)
                buf[i, sl] = buf[i, sl] + pos_v[i, sl]
            return c
        lax.fori_loop(0, SEQ, body, 0)

    def pair_body(g, c):
        s0, s1 = 2 * g, 2 * g + 1

        @pl.when(g > 0)
        def _():
            wait_writeout(row_a, wsem_a)

        fire_gathers(s0, row_a, gsem_a)

        @pl.when(g > 0)
        def _():
            wait_writeout(row_b, wsem_b)

        fire_gathers(s1, row_b, gsem_b)

        wait_gathers(row_a, gsem_a)
        add_pos(row_a)
        fire_writeout(s0, row_a, wsem_a)

        wait_gathers(row_b, gsem_b)
        add_pos(row_b)
        fire_writeout(s1, row_b, wsem_b)
        return c

    lax.fori_loop(0, SPW // 2, pair_body, 0)
    wait_writeout(row_a, wsem_a)
    wait_writeout(row_b, wsem_b)


def kernel(input, tok_table, pos_table):
    idx = input.astype(jnp.int32).reshape(BATCH * SEQ // CH, CH)

    mesh = plsc.VectorSubcoreMesh(
        core_axis_name="c", subcore_axis_name="s", num_cores=NC, num_subcores=NS
    )
    emb = functools.partial(
        pl.kernel,
        out_type=jax.ShapeDtypeStruct((BATCH * SEQ, EMBED), jnp.float32),
        mesh=mesh,
        scratch_types=[
            pltpu.VMEM((SEQ, EMBED), jnp.float32),
            pltpu.VMEM((SPW * NCH, CH), jnp.int32),
            pltpu.VMEM((SEQ, EMBED), jnp.float32),
            pltpu.VMEM((SEQ, EMBED), jnp.float32),
            pltpu.SemaphoreType.DMA,
            pltpu.SemaphoreType.DMA,
            pltpu.SemaphoreType.DMA,
            pltpu.SemaphoreType.DMA,
        ],
    )(_emb_body)
    out = emb(idx, tok_table, pos_table)
    return out.reshape(BATCH, SEQ, EMBED)


# speedup vs baseline: 14.1873x; 1.1989x over previous
import functools

import jax
import jax.numpy as jnp
from jax import lax
from jax.experimental import pallas as pl
from jax.experimental.pallas import tpu as pltpu
from jax.experimental.pallas import tpu_sc as plsc

NC = 2
NS = 16
NW = NC * NS

BATCH = 1024
SEQ = 200
EMBED = 128
LANES = 16
CH = 40
NCH = SEQ // CH
SPW = BATCH // NW


def _emb_body(idx_hbm, tok_hbm, pos_hbm, out_hbm,
              pos_v, idx_v, row_a, row_b, row_c,
              gsem_a, gsem_b, gsem_c, wsem_a, wsem_b, wsem_c):
    wid = lax.axis_index("s") * NC + lax.axis_index("c")

    pltpu.sync_copy(pos_hbm.at[pl.ds(0, SEQ)], pos_v)
    pltpu.sync_copy(idx_hbm.at[pl.ds(wid * SPW * NCH, SPW * NCH)], idx_v)

    def fire_gathers(s, buf, sem):
        for j in range(NCH):
            pltpu.async_copy(
                tok_hbm.at[idx_v.at[s * NCH + j]],
                buf.at[pl.ds(j * CH, CH)],
                sem,
            )

    def wait_gathers(buf, sem):
        pltpu.make_async_copy(tok_hbm.at[pl.ds(0, SEQ)], buf, sem).wait()

    def fire_writeout(s, buf, sem):
        base = (wid * SPW + s) * SEQ
        pltpu.async_copy(buf, out_hbm.at[pl.ds(base, SEQ)], sem)

    def wait_writeout(buf, sem):
        pltpu.make_async_copy(buf, out_hbm.at[pl.ds(0, SEQ)], sem).wait()

    def add_pos(buf):
        def body(i, c):
            for j in range(EMBED // LANES):
                sl = pl.ds(j * LANES, LANES)
                buf[i, sl] = buf[i, sl] + pos_v[i, sl]
            return c
        lax.fori_loop(0, SEQ, body, 0)

    bufs = (row_a, row_b, row_c)
    gsems = (gsem_a, gsem_b, gsem_c)
    wsems = (wsem_a, wsem_b, wsem_c)

    def stage(s, r, wait_prev=True, fire_next=True):
        cur, nxt = r % 3, (r + 1) % 3
        if fire_next:
            if wait_prev:
                wait_writeout(bufs[nxt], wsems[nxt])
            fire_gathers(s + 1, bufs[nxt], gsems[nxt])
        wait_gathers(bufs[cur], gsems[cur])
        add_pos(bufs[cur])
        fire_writeout(s, bufs[cur], wsems[cur])

    fire_gathers(0, bufs[0], gsems[0])
    stage(0, 0, wait_prev=False)
    stage(1, 1, wait_prev=False)

    def tri_body(g, c):
        s = 3 * g + 2
        stage(s, 2)
        stage(s + 1, 0)
        stage(s + 2, 1)
        return c

    lax.fori_loop(0, (SPW - 5) // 3, tri_body, 0)

    stage(SPW - 3, 2)
    stage(SPW - 2, 0)
    stage(SPW - 1, 1, fire_next=False)

    wait_writeout(bufs[2], wsems[2])
    wait_writeout(bufs[0], wsems[0])
    wait_writeout(bufs[1], wsems[1])


def kernel(input, tok_table, pos_table):
    idx = input.astype(jnp.int32).reshape(BATCH * SEQ // CH, CH)

    mesh = plsc.VectorSubcoreMesh(
        core_axis_name="c", subcore_axis_name="s", num_cores=NC, num_subcores=NS
    )
    emb = functools.partial(
        pl.kernel,
        out_type=jax.ShapeDtypeStruct((BATCH * SEQ, EMBED), jnp.float32),
        mesh=mesh,
        scratch_types=[
            pltpu.VMEM((SEQ, EMBED), jnp.float32),
            pltpu.VMEM((SPW * NCH, CH), jnp.int32),
            pltpu.VMEM((SEQ, EMBED), jnp.float32),
            pltpu.VMEM((SEQ, EMBED), jnp.float32),
            pltpu.VMEM((SEQ, EMBED), jnp.float32),
            pltpu.SemaphoreType.DMA,
            pltpu.SemaphoreType.DMA,
            pltpu.SemaphoreType.DMA,
            pltpu.SemaphoreType.DMA,
            pltpu.SemaphoreType.DMA,
            pltpu.SemaphoreType.DMA,
        ],
    )(_emb_body)
    out = emb(idx, tok_table, pos_table)
    return out.reshape(BATCH, SEQ, EMBED)


# speedup vs baseline: 14.5360x; 1.0246x over previous
import functools

import jax
import jax.numpy as jnp
from jax import lax
from jax.experimental import pallas as pl
from jax.experimental.pallas import tpu as pltpu
from jax.experimental.pallas import tpu_sc as plsc

NC = 2
NS = 16
NW = NC * NS

BATCH = 1024
SEQ = 200
EMBED = 128
LANES = 16
CH = 100
NCH = SEQ // CH
SPW = BATCH // NW


def _emb_body(idx_hbm, tok_hbm, pos_hbm, out_hbm,
              pos_v, idx_v, row_a, row_b, row_c,
              gsem_a, gsem_b, gsem_c, wsem_a, wsem_b, wsem_c):
    wid = lax.axis_index("s") * NC + lax.axis_index("c")

    pltpu.sync_copy(pos_hbm.at[pl.ds(0, SEQ)], pos_v)
    pltpu.sync_copy(idx_hbm.at[pl.ds(wid * SPW * NCH, SPW * NCH)], idx_v)

    def fire_gathers(s, buf, sem):
        for j in range(NCH):
            pltpu.async_copy(
                tok_hbm.at[idx_v.at[s * NCH + j]],
                buf.at[pl.ds(j * CH, CH)],
                sem,
            )

    def wait_gathers(buf, sem):
        pltpu.make_async_copy(tok_hbm.at[pl.ds(0, SEQ)], buf, sem).wait()

    def fire_writeout(s, buf, sem):
        base = (wid * SPW + s) * SEQ
        pltpu.async_copy(buf, out_hbm.at[pl.ds(base, SEQ)], sem)

    def wait_writeout(buf, sem):
        pltpu.make_async_copy(buf, out_hbm.at[pl.ds(0, SEQ)], sem).wait()

    def add_pos(buf):
        def body(i, c):
            for j in range(EMBED // LANES):
                sl = pl.ds(j * LANES, LANES)
                buf[i, sl] = buf[i, sl] + pos_v[i, sl]
            return c
        lax.fori_loop(0, SEQ, body, 0)

    bufs = (row_a, row_b, row_c)
    gsems = (gsem_a, gsem_b, gsem_c)
    wsems = (wsem_a, wsem_b, wsem_c)

    def stage(s, r, wait_prev=True, fire_next=True):
        cur, nxt = r % 3, (r + 1) % 3
        if fire_next:
            if wait_prev:
                wait_writeout(bufs[nxt], wsems[nxt])
            fire_gathers(s + 1, bufs[nxt], gsems[nxt])
        wait_gathers(bufs[cur], gsems[cur])
        add_pos(bufs[cur])
        fire_writeout(s, bufs[cur], wsems[cur])

    fire_gathers(0, bufs[0], gsems[0])
    stage(0, 0, wait_prev=False)
    stage(1, 1, wait_prev=False)

    def tri_body(g, c):
        s = 3 * g + 2
        stage(s, 2)
        stage(s + 1, 0)
        stage(s + 2, 1)
        return c

    lax.fori_loop(0, (SPW - 5) // 3, tri_body, 0)

    stage(SPW - 3, 2)
    stage(SPW - 2, 0)
    stage(SPW - 1, 1, fire_next=False)

    wait_writeout(bufs[2], wsems[2])
    wait_writeout(bufs[0], wsems[0])
    wait_writeout(bufs[1], wsems[1])


def kernel(input, tok_table, pos_table):
    idx = input.astype(jnp.int32).reshape(BATCH * SEQ // CH, CH)

    mesh = plsc.VectorSubcoreMesh(
        core_axis_name="c", subcore_axis_name="s", num_cores=NC, num_subcores=NS
    )
    emb = functools.partial(
        pl.kernel,
        out_type=jax.ShapeDtypeStruct((BATCH * SEQ, EMBED), jnp.float32),
        mesh=mesh,
        scratch_types=[
            pltpu.VMEM((SEQ, EMBED), jnp.float32),
            pltpu.VMEM((SPW * NCH, CH), jnp.int32),
            pltpu.VMEM((SEQ, EMBED), jnp.float32),
            pltpu.VMEM((SEQ, EMBED), jnp.float32),
            pltpu.VMEM((SEQ, EMBED), jnp.float32),
            pltpu.SemaphoreType.DMA,
            pltpu.SemaphoreType.DMA,
            pltpu.SemaphoreType.DMA,
            pltpu.SemaphoreType.DMA,
            pltpu.SemaphoreType.DMA,
            pltpu.SemaphoreType.DMA,
        ],
    )(_emb_body)
    out = emb(idx, tok_table, pos_table)
    return out.reshape(BATCH, SEQ, EMBED)
